# baseline (device time: 379775 ns/iter reference)
import jax
import jax.numpy as jnp
from jax import lax
from jax.experimental import pallas as pl
from jax.experimental.pallas import tpu as pltpu

N_DEV = 16
M_PER = 256

RING = [0, 4, 8, 12, 15, 11, 7, 3, 2, 6, 10, 14, 13, 9, 5, 1]
INV = [0] * N_DEV
for _p, _d in enumerate(RING):
    INV[_d] = _p


def kernel(x, w_mat, scale_x, scale_w):
    m, k_per = x.shape
    _, n = w_mat.shape

    my = lax.axis_index("i")
    ring_arr = jnp.array(RING, dtype=jnp.int32)
    inv_arr = jnp.array(INV, dtype=jnp.int32)
    r = inv_arr[my]
    nbrs = jnp.stack([ring_arr[(r + 1) % N_DEV], ring_arr[(r - 1) % N_DEV]])
    cids = ring_arr[(r - 1 - jnp.arange(N_DEV, dtype=jnp.int32)) % N_DEV]

    def body(nbr_ref, cid_ref, x_ref, w_ref, sx_ref, sw_ref, out_ref,
             comm_ref, send_sems, recv_sems):
        right = nbr_ref[0]
        left = nbr_ref[1]

        barrier_sem = pltpu.get_barrier_semaphore()
        for nbr in (left, right):
            pl.semaphore_signal(
                barrier_sem, inc=1,
                device_id=(nbr,), device_id_type=pl.DeviceIdType.MESH,
            )
        pl.semaphore_wait(barrier_sem, 2)

        def partial(cid):
            xs = x_ref[pl.ds(cid * M_PER, M_PER), :]
            return lax.dot_general(
                xs, w_ref[:, :], (((1,), (0,)), ((), ())),
                preferred_element_type=jnp.float32,
            )

        comm_ref[0, :, :] = partial(cid_ref[0])

        for s in range(N_DEV - 1):
            send_slot = s % 2
            recv_slot = (s + 1) % 2
            rdma = pltpu.make_async_remote_copy(
                src_ref=comm_ref.at[send_slot],
                dst_ref=comm_ref.at[recv_slot],
                send_sem=send_sems.at[send_slot],
                recv_sem=recv_sems.at[recv_slot],
                device_id=(right,),
                device_id_type=pl.DeviceIdType.MESH,
            )
            rdma.start()
            part = partial(cid_ref[s + 1])
            rdma.wait()
            if s < N_DEV - 2:
                comm_ref[recv_slot, :, :] = comm_ref[recv_slot, :, :] + part
            else:
                acc = comm_ref[recv_slot, :, :] + part
                y = acc * (sx_ref[0] * sw_ref[0])
                out_ref[:, :] = y * jax.nn.sigmoid(jnp.clip(y, -60.0, 60.0))

    return pl.pallas_call(
        body,
        out_shape=jax.ShapeDtypeStruct((M_PER, n), jnp.float32),
        in_specs=[
            pl.BlockSpec(memory_space=pltpu.SMEM),
            pl.BlockSpec(memory_space=pltpu.SMEM),
            pl.BlockSpec(memory_space=pltpu.VMEM),
            pl.BlockSpec(memory_space=pltpu.VMEM),
            pl.BlockSpec(memory_space=pltpu.SMEM),
            pl.BlockSpec(memory_space=pltpu.SMEM),
        ],
        out_specs=pl.BlockSpec(memory_space=pltpu.VMEM),
        scratch_shapes=[
            pltpu.VMEM((2, M_PER, n), jnp.float32),
            pltpu.SemaphoreType.DMA((2,)),
            pltpu.SemaphoreType.DMA((2,)),
        ],
        compiler_params=pltpu.CompilerParams(collective_id=0),
    )(nbrs, cids, x, w_mat, scale_x, scale_w)


# device time: 131316 ns/iter; 2.8921x vs baseline; 2.8921x over previous
import jax
import jax.numpy as jnp
from jax import lax
from jax.experimental import pallas as pl
from jax.experimental.pallas import tpu as pltpu

N_DEV = 16
M_PER = 256
COMM_DTYPE = jnp.bfloat16

RING = [0, 4, 8, 12, 15, 11, 7, 3, 2, 6, 10, 14, 13, 9, 5, 1]
INV = [0] * N_DEV
for _p, _d in enumerate(RING):
    INV[_d] = _p


def kernel(x, w_mat, scale_x, scale_w):
    m, k_per = x.shape
    _, n = w_mat.shape
    n_half = n // 2

    my = lax.axis_index("i")
    ring_arr = jnp.array(RING, dtype=jnp.int32)
    inv_arr = jnp.array(INV, dtype=jnp.int32)
    r = inv_arr[my]
    nbrs = jnp.stack([ring_arr[(r + 1) % N_DEV], ring_arr[(r - 1) % N_DEV]])
    steps = jnp.arange(N_DEV, dtype=jnp.int32)
    cids_a = ring_arr[(r - 1 - steps) % N_DEV]
    cids_b = ring_arr[(r + 1 + steps) % N_DEV]

    def body(nbr_ref, cida_ref, cidb_ref, x_ref, w_ref, sx_ref, sw_ref,
             out_ref, comm_a, comm_b, send_a, recv_a, send_b, recv_b):
        right = nbr_ref[0]
        left = nbr_ref[1]

        barrier_sem = pltpu.get_barrier_semaphore()
        for nbr in (left, right):
            pl.semaphore_signal(
                barrier_sem, inc=1,
                device_id=(nbr,), device_id_type=pl.DeviceIdType.MESH,
            )
        pl.semaphore_wait(barrier_sem, 2)

        def partial(cid, col0):
            xs = x_ref[pl.ds(cid * M_PER, M_PER), :]
            ws = w_ref[:, pl.ds(col0, n_half)]
            return lax.dot_general(
                xs, ws, (((1,), (0,)), ((), ())),
                preferred_element_type=jnp.float32,
            )

        comm_a[0, :, :] = partial(cida_ref[0], 0).astype(COMM_DTYPE)
        comm_b[0, :, :] = partial(cidb_ref[0], n_half).astype(COMM_DTYPE)

        for s in range(N_DEV - 1):
            send_slot = s % 2
            recv_slot = (s + 1) % 2
            rdma_a = pltpu.make_async_remote_copy(
                src_ref=comm_a.at[send_slot],
                dst_ref=comm_a.at[recv_slot],
                send_sem=send_a.at[send_slot],
                recv_sem=recv_a.at[recv_slot],
                device_id=(right,),
                device_id_type=pl.DeviceIdType.MESH,
            )
            rdma_b = pltpu.make_async_remote_copy(
                src_ref=comm_b.at[send_slot],
                dst_ref=comm_b.at[recv_slot],
                send_sem=send_b.at[send_slot],
                recv_sem=recv_b.at[recv_slot],
                device_id=(left,),
                device_id_type=pl.DeviceIdType.MESH,
            )
            rdma_a.start()
            rdma_b.start()
            part_a = partial(cida_ref[s + 1], 0)
            part_b = partial(cidb_ref[s + 1], n_half)
            rdma_a.wait()
            rdma_b.wait()
            if s < N_DEV - 2:
                comm_a[recv_slot, :, :] = (
                    comm_a[recv_slot, :, :].astype(jnp.float32) + part_a
                ).astype(COMM_DTYPE)
                comm_b[recv_slot, :, :] = (
                    comm_b[recv_slot, :, :].astype(jnp.float32) + part_b
                ).astype(COMM_DTYPE)
            else:
                scale = sx_ref[0] * sw_ref[0]
                acc_a = comm_a[recv_slot, :, :].astype(jnp.float32) + part_a
                acc_b = comm_b[recv_slot, :, :].astype(jnp.float32) + part_b
                ya = acc_a * scale
                yb = acc_b * scale
                out_ref[:, 0:n_half] = ya * jax.nn.sigmoid(
                    jnp.clip(ya, -60.0, 60.0))
                out_ref[:, n_half:n] = yb * jax.nn.sigmoid(
                    jnp.clip(yb, -60.0, 60.0))

    return pl.pallas_call(
        body,
        out_shape=jax.ShapeDtypeStruct((M_PER, n), jnp.float32),
        in_specs=[
            pl.BlockSpec(memory_space=pltpu.SMEM),
            pl.BlockSpec(memory_space=pltpu.SMEM),
            pl.BlockSpec(memory_space=pltpu.SMEM),
            pl.BlockSpec(memory_space=pltpu.VMEM),
            pl.BlockSpec(memory_space=pltpu.VMEM),
            pl.BlockSpec(memory_space=pltpu.SMEM),
            pl.BlockSpec(memory_space=pltpu.SMEM),
        ],
        out_specs=pl.BlockSpec(memory_space=pltpu.VMEM),
        scratch_shapes=[
            pltpu.VMEM((2, M_PER, n_half), COMM_DTYPE),
            pltpu.VMEM((2, M_PER, n_half), COMM_DTYPE),
            pltpu.SemaphoreType.DMA((2,)),
            pltpu.SemaphoreType.DMA((2,)),
            pltpu.SemaphoreType.DMA((2,)),
            pltpu.SemaphoreType.DMA((2,)),
        ],
        compiler_params=pltpu.CompilerParams(collective_id=0),
    )(nbrs, cids_a, cids_b, x, w_mat, scale_x, scale_w)


# device time: 101308 ns/iter; 3.7487x vs baseline; 1.2962x over previous
import jax
import jax.numpy as jnp
from jax import lax
from jax.experimental import pallas as pl
from jax.experimental.pallas import tpu as pltpu

N_DEV = 16
M_PER = 256
COMM_DTYPE = jnp.bfloat16

RING = [0, 4, 8, 12, 15, 11, 7, 3, 2, 6, 10, 14, 13, 9, 5, 1]
INV = [0] * N_DEV
for _p, _d in enumerate(RING):
    INV[_d] = _p


def kernel(x, w_mat, scale_x, scale_w):
    m, k_per = x.shape
    _, n = w_mat.shape
    n_sub = n // 4

    my = lax.axis_index("i")
    ring_arr = jnp.array(RING, dtype=jnp.int32)
    inv_arr = jnp.array(INV, dtype=jnp.int32)
    r = inv_arr[my]
    nbrs = jnp.stack([ring_arr[(r + 1) % N_DEV], ring_arr[(r - 1) % N_DEV]])
    steps = jnp.arange(N_DEV, dtype=jnp.int32)
    cids_a = ring_arr[(r - 1 - steps) % N_DEV]
    cids_b = ring_arr[(r + 1 + steps) % N_DEV]

    def body(nbr_ref, cida_ref, cidb_ref, x_ref, w_ref, sx_ref, sw_ref,
             out_ref, c0, c1, c2, c3,
             ss0, rs0, ss1, rs1, ss2, rs2, ss3, rs3):
        right = nbr_ref[0]
        left = nbr_ref[1]

        barrier_sem = pltpu.get_barrier_semaphore()
        for nbr in (left, right):
            pl.semaphore_signal(
                barrier_sem, inc=1,
                device_id=(nbr,), device_id_type=pl.DeviceIdType.MESH,
            )
        pl.semaphore_wait(barrier_sem, 2)

        subs = [
            (c0, ss0, rs0, right, 0 * n_sub, cida_ref),
            (c2, ss2, rs2, left, 2 * n_sub, cidb_ref),
            (c1, ss1, rs1, right, 1 * n_sub, cida_ref),
            (c3, ss3, rs3, left, 3 * n_sub, cidb_ref),
        ]

        def partial(cid, col0):
            xs = x_ref[pl.ds(cid * M_PER, M_PER), :]
            ws = w_ref[:, col0:col0 + n_sub]
            return lax.dot_general(
                xs, ws, (((1,), (0,)), ((), ())),
                preferred_element_type=jnp.float32,
            )

        def make_rdma(comm, ssem, rsem, dev, t):
            return pltpu.make_async_remote_copy(
                src_ref=comm.at[t % 2],
                dst_ref=comm.at[(t + 1) % 2],
                send_sem=ssem.at[t % 2],
                recv_sem=rsem.at[(t + 1) % 2],
                device_id=(dev,),
                device_id_type=pl.DeviceIdType.MESH,
            )

        inflight = []
        for comm, ssem, rsem, dev, col0, cids in subs:
            comm[0, :, :] = partial(cids[0], col0).astype(COMM_DTYPE)
            rdma = make_rdma(comm, ssem, rsem, dev, 0)
            rdma.start()
            inflight.append(rdma)

        for t in range(1, N_DEV - 1):
            for i, (comm, ssem, rsem, dev, col0, cids) in enumerate(subs):
                part = partial(cids[t], col0)
                inflight[i].wait()
                slot = t % 2
                comm[slot, :, :] = (
                    comm[slot, :, :].astype(jnp.float32) + part
                ).astype(COMM_DTYPE)
                rdma = make_rdma(comm, ssem, rsem, dev, t)
                rdma.start()
                inflight[i] = rdma

        scale = sx_ref[0] * sw_ref[0]
        for i, (comm, ssem, rsem, dev, col0, cids) in enumerate(subs):
            part = partial(cids[N_DEV - 1], col0)
            inflight[i].wait()
            acc = comm[(N_DEV - 1) % 2, :, :].astype(jnp.float32) + part
            y = acc * scale
            out_ref[:, col0:col0 + n_sub] = y * jax.nn.sigmoid(
                jnp.clip(y, -60.0, 60.0))

    return pl.pallas_call(
        body,
        out_shape=jax.ShapeDtypeStruct((M_PER, n), jnp.float32),
        in_specs=[
            pl.BlockSpec(memory_space=pltpu.SMEM),
            pl.BlockSpec(memory_space=pltpu.SMEM),
            pl.BlockSpec(memory_space=pltpu.SMEM),
            pl.BlockSpec(memory_space=pltpu.VMEM),
            pl.BlockSpec(memory_space=pltpu.VMEM),
            pl.BlockSpec(memory_space=pltpu.SMEM),
            pl.BlockSpec(memory_space=pltpu.SMEM),
        ],
        out_specs=pl.BlockSpec(memory_space=pltpu.VMEM),
        scratch_shapes=[
            pltpu.VMEM((2, M_PER, n_sub), COMM_DTYPE),
            pltpu.VMEM((2, M_PER, n_sub), COMM_DTYPE),
            pltpu.VMEM((2, M_PER, n_sub), COMM_DTYPE),
            pltpu.VMEM((2, M_PER, n_sub), COMM_DTYPE),
            pltpu.SemaphoreType.DMA((2,)),
            pltpu.SemaphoreType.DMA((2,)),
            pltpu.SemaphoreType.DMA((2,)),
            pltpu.SemaphoreType.DMA((2,)),
            pltpu.SemaphoreType.DMA((2,)),
            pltpu.SemaphoreType.DMA((2,)),
            pltpu.SemaphoreType.DMA((2,)),
            pltpu.SemaphoreType.DMA((2,)),
        ],
        compiler_params=pltpu.CompilerParams(collective_id=0),
    )(nbrs, cids_a, cids_b, x, w_mat, scale_x, scale_w)


# device time: 101026 ns/iter; 3.7592x vs baseline; 1.0028x over previous
import jax
import jax.numpy as jnp
from jax import lax
from jax.experimental import pallas as pl
from jax.experimental.pallas import tpu as pltpu

N_DEV = 16
M_PER = 256
COMM_DTYPE = jnp.bfloat16
SUBS_PER_DIR = 4

RING = [0, 4, 8, 12, 15, 11, 7, 3, 2, 6, 10, 14, 13, 9, 5, 1]
INV = [0] * N_DEV
for _p, _d in enumerate(RING):
    INV[_d] = _p


def kernel(x, w_mat, scale_x, scale_w):
    m, k_per = x.shape
    _, n = w_mat.shape
    n_half = n // 2
    n_sub = n_half // SUBS_PER_DIR
    n_subs = 2 * SUBS_PER_DIR

    my = lax.axis_index("i")
    ring_arr = jnp.array(RING, dtype=jnp.int32)
    inv_arr = jnp.array(INV, dtype=jnp.int32)
    r = inv_arr[my]
    nbrs = jnp.stack([ring_arr[(r + 1) % N_DEV], ring_arr[(r - 1) % N_DEV]])
    steps = jnp.arange(N_DEV, dtype=jnp.int32)
    cids_a = ring_arr[(r - 1 - steps) % N_DEV]
    cids_b = ring_arr[(r + 1 + steps) % N_DEV]

    def body(nbr_ref, cida_ref, cidb_ref, x_ref, w_ref, sx_ref, sw_ref,
             out_ref, *scr):
        comms = scr[:n_subs]
        sems = scr[n_subs:]
        right = nbr_ref[0]
        left = nbr_ref[1]

        barrier_sem = pltpu.get_barrier_semaphore()
        for nbr in (left, right):
            pl.semaphore_signal(
                barrier_sem, inc=1,
                device_id=(nbr,), device_id_type=pl.DeviceIdType.MESH,
            )
        pl.semaphore_wait(barrier_sem, 2)

        subs = []
        for j in range(SUBS_PER_DIR):
            subs.append((comms[2 * j], sems[4 * j], sems[4 * j + 1],
                         right, j * n_sub, cida_ref))
            subs.append((comms[2 * j + 1], sems[4 * j + 2], sems[4 * j + 3],
                         left, n_half + j * n_sub, cidb_ref))

        def partial(cid, col0):
            xs = x_ref[pl.ds(cid * M_PER, M_PER), :]
            ws = w_ref[:, col0:col0 + n_sub]
            return lax.dot_general(
                xs, ws, (((1,), (0,)), ((), ())),
                preferred_element_type=jnp.float32,
            )

        def make_rdma(comm, ssem, rsem, dev, t):
            return pltpu.make_async_remote_copy(
                src_ref=comm.at[t % 2],
                dst_ref=comm.at[(t + 1) % 2],
                send_sem=ssem.at[t % 2],
                recv_sem=rsem.at[(t + 1) % 2],
                device_id=(dev,),
                device_id_type=pl.DeviceIdType.MESH,
            )

        inflight = []
        for comm, ssem, rsem, dev, col0, cids in subs:
            comm[0, :, :] = partial(cids[0], col0).astype(COMM_DTYPE)
            rdma = make_rdma(comm, ssem, rsem, dev, 0)
            rdma.start()
            inflight.append(rdma)

        for t in range(1, N_DEV - 1):
            for i, (comm, ssem, rsem, dev, col0, cids) in enumerate(subs):
                part = partial(cids[t], col0)
                inflight[i].wait()
                slot = t % 2
                comm[slot, :, :] = (
                    comm[slot, :, :].astype(jnp.float32) + part
                ).astype(COMM_DTYPE)
                rdma = make_rdma(comm, ssem, rsem, dev, t)
                rdma.start()
                inflight[i] = rdma

        scale = sx_ref[0] * sw_ref[0]
        for i, (comm, ssem, rsem, dev, col0, cids) in enumerate(subs):
            part = partial(cids[N_DEV - 1], col0)
            inflight[i].wait()
            acc = comm[(N_DEV - 1) % 2, :, :].astype(jnp.float32) + part
            y = acc * scale
            out_ref[:, col0:col0 + n_sub] = y * jax.nn.sigmoid(
                jnp.clip(y, -60.0, 60.0))

    scratch = [pltpu.VMEM((2, M_PER, n_sub), COMM_DTYPE)
               for _ in range(n_subs)]
    scratch += [pltpu.SemaphoreType.DMA((2,)) for _ in range(2 * n_subs)]

    return pl.pallas_call(
        body,
        out_shape=jax.ShapeDtypeStruct((M_PER, n), jnp.float32),
        in_specs=[
            pl.BlockSpec(memory_space=pltpu.SMEM),
            pl.BlockSpec(memory_space=pltpu.SMEM),
            pl.BlockSpec(memory_space=pltpu.SMEM),
            pl.BlockSpec(memory_space=pltpu.VMEM),
            pl.BlockSpec(memory_space=pltpu.VMEM),
            pl.BlockSpec(memory_space=pltpu.SMEM),
            pl.BlockSpec(memory_space=pltpu.SMEM),
        ],
        out_specs=pl.BlockSpec(memory_space=pltpu.VMEM),
        scratch_shapes=scratch,
        compiler_params=pltpu.CompilerParams(collective_id=0),
    )(nbrs, cids_a, cids_b, x, w_mat, scale_x, scale_w)
